# baseline (speedup 1.0000x reference)
import functools

import jax
import jax.numpy as jnp
from jax import lax
from jax.experimental import pallas as pl
from jax.experimental.pallas import tpu as pltpu
from jax.experimental.pallas import tpu_sc as plsc

NC = 2
NS = 16
NW = NC * NS
R = 8


def _make_gather(B, H, V, D):
    rows_per_w = B // NW
    n_sch = rows_per_w // R
    n_half = n_sch // 2
    assert n_sch % 2 == 0 and n_sch * R == rows_per_w
    mesh = plsc.VectorSubcoreMesh(core_axis_name="c", subcore_axis_name="s")

    @functools.partial(
        pl.kernel,
        mesh=mesh,
        out_type=jax.ShapeDtypeStruct((B, H, D), jnp.float32),
        scratch_types=[
            pltpu.VMEM((rows_per_w, H), jnp.int32),
            pltpu.VMEM((2, R, H, D), jnp.float32),
            pltpu.SemaphoreType.DMA,
            pltpu.SemaphoreType.DMA,
        ],
        compiler_params=pltpu.CompilerParams(use_tc_tiling_on_sc=False),
    )
    def k(idx_hbm, table_hbm, out_hbm, idx_v, buf, sem_g, sem_s):
        wid = lax.axis_index("s") * NC + lax.axis_index("c")
        row0 = wid * rows_per_w
        pltpu.sync_copy(idx_hbm.at[pl.ds(row0, rows_per_w)], idx_v)

        def fire(s, b):
            for t in range(R):
                pltpu.async_copy(
                    table_hbm.at[idx_v.at[s * R + t]],
                    buf.at[b, t],
                    sem_g,
                )

        def wait_gathers(b):
            pltpu.make_async_copy(
                out_hbm.at[pl.ds(row0, R)],
                buf.at[b],
                sem_g,
            ).wait()

        def store(s, b):
            pltpu.async_copy(
                buf.at[b], out_hbm.at[pl.ds(row0 + s * R, R)], sem_s
            )

        def wait_store(s, b):
            pltpu.make_async_copy(
                buf.at[b], out_hbm.at[pl.ds(row0 + s * R, R)], sem_s
            ).wait()

        fire(0, 0)

        def body(i, carry):
            s0 = i * 2
            wait_gathers(0)

            @pl.when(i > 0)
            def _():
                wait_store(s0 - 1, 1)

            fire(s0 + 1, 1)
            store(s0, 0)
            wait_gathers(1)

            @pl.when(i < n_half - 1)
            def _():
                wait_store(s0, 0)
                fire(s0 + 2, 0)

            store(s0 + 1, 1)
            return carry

        lax.fori_loop(0, n_half, body, 0, unroll=False)
        wait_store(n_sch - 2, 0)
        wait_store(n_sch - 1, 1)

    return k


def kernel(x, W):
    B, H = x.shape
    V, D = W.shape
    return _make_gather(B, H, V, D)(x.astype(jnp.int32), W)
